# Initial kernel scaffold; baseline (speedup 1.0000x reference)
import jax
import jax.numpy as jnp
from jax.experimental import pallas as pl


def kernel(x_l, edge_index_l, edge_weight_l, x_s, edge_index_s, edge_weight_s, batch_index_l, batch_index_s, W1a, b1a, W1b, b1b, Wl1, bl1, Wl2, bl2, gamma, beta, Wo, bo):
    raise NotImplementedError("write your pallas kernel here")


# speedup vs baseline: 11.9765x; 11.9765x over previous
import functools

import jax
import jax.numpy as jnp
from jax import lax
from jax.experimental import pallas as pl
from jax.experimental.pallas import tpu as pltpu
from jax.experimental.pallas import tpu_sc as plsc

N = 10000
F = 128
G = 128
NC = 2
NS = 16
NW = NC * NS
C = 128
NROW = N // 16
NP = 10112
SLAB = NP // NS

_mesh = plsc.VectorSubcoreMesh(core_axis_name="c", subcore_axis_name="s")
_sc_params = pltpu.CompilerParams(needs_layout_passes=False)


def _fastrsqrt(d):
    xi = lax.bitcast_convert_type(d, jnp.int32)
    yi = jnp.int32(0x5F3759DF) - lax.shift_right_logical(xi, 1)
    y = lax.bitcast_convert_type(yi, jnp.float32)
    for _ in range(3):
        y = y * (1.5 - 0.5 * d * y * y)
    return jnp.where(d > 0, y, 0.0)


@functools.partial(
    pl.kernel,
    out_type=(
        jax.ShapeDtypeStruct((NC * NP,), jnp.float32),
        jax.ShapeDtypeStruct((NC * NP,), jnp.float32),
    ),
    mesh=_mesh,
    compiler_params=_sc_params,
    scratch_types=[
        pltpu.VMEM_SHARED((NP,), jnp.float32),
        pltpu.VMEM_SHARED((NP,), jnp.float32),
        pltpu.VMEM((C,), jnp.int32),
        pltpu.VMEM((C,), jnp.float32),
    ],
)
def _deg_kernel(dst_l, ew_l, dst_s, ew_s, z1, degp_l, degp_s,
                acc_l, acc_s, dstbuf, ewbuf):
    c = lax.axis_index("c")
    s = lax.axis_index("s")
    w = s * NC + c

    @pl.when(s == 0)
    def _():
        pltpu.sync_copy(z1, acc_l)
        pltpu.sync_copy(z1, acc_s)

    plsc.subcore_barrier()

    def run(dst_hbm, ew_hbm, nk, acc):
        def body(j, _):
            k = w + j * NW

            @pl.when(k < nk)
            def _():
                pltpu.sync_copy(dst_hbm.at[pl.ds(k * C, C)], dstbuf)
                pltpu.sync_copy(ew_hbm.at[pl.ds(k * C, C)], ewbuf)
                pltpu.sync_copy(ewbuf, acc.at[dstbuf], add=True)

            return 0

        lax.fori_loop(0, (nk + NW - 1) // NW, body, 0)

    run(dst_l, ew_l, 320000 // C, acc_l)
    run(dst_s, ew_s, 160000 // C, acc_s)
    plsc.subcore_barrier()

    @pl.when(s == 0)
    def _():
        pltpu.sync_copy(acc_l, degp_l.at[pl.ds(c * NP, NP)])
        pltpu.sync_copy(acc_s, degp_s.at[pl.ds(c * NP, NP)])


def _mm_body(xl_ref, wa_ref, xs_ref, wb_ref, hl_ref, hs_ref):
    hl_ref[...] = jnp.dot(xl_ref[...], wa_ref[...],
                          preferred_element_type=jnp.float32)
    hs_ref[...] = jnp.dot(xs_ref[...], wb_ref[...],
                          preferred_element_type=jnp.float32)


@functools.partial(
    pl.kernel,
    out_type=(
        jax.ShapeDtypeStruct((NC, NP, F), jnp.float32),
        jax.ShapeDtypeStruct((NC, NP, F), jnp.float32),
    ),
    mesh=_mesh,
    compiler_params=_sc_params,
    scratch_types=[
        pltpu.VMEM_SHARED((NP, F), jnp.float32),
        pltpu.VMEM((NP,), jnp.float32),
        pltpu.VMEM((NP,), jnp.float32),
        pltpu.VMEM((NP,), jnp.float32),
        pltpu.VMEM((C,), jnp.int32),
        pltpu.VMEM((C,), jnp.int32),
        pltpu.VMEM((C,), jnp.float32),
        pltpu.VMEM((C,), jnp.float32),
        pltpu.VMEM((C, F), jnp.float32),
        pltpu.SemaphoreType.DMA,
    ],
)
def _msg_kernel(h_l, h_s, degp_l, degp_s,
                src_l, dst_l, ew_l, src_s, dst_s, ew_s, z128,
                accp_l, accp_s,
                acc, dp0, dp1, dinvbuf, srcbuf, dstbuf, ewbuf, normbuf,
                rows, sem):
    c = lax.axis_index("c")
    s = lax.axis_index("s")
    w = s * NC + c

    def compute_dinv(degp):
        pltpu.sync_copy(degp.at[pl.ds(0, NP)], dp0)
        pltpu.sync_copy(degp.at[pl.ds(NP, NP)], dp1)

        def db(i, _):
            d = dp0[pl.ds(i * 16, 16)] + dp1[pl.ds(i * 16, 16)]
            dinvbuf[pl.ds(i * 16, 16)] = _fastrsqrt(d)
            return 0

        lax.fori_loop(0, NROW, db, 0)

    def graph_pass(h_hbm, src_hbm, dst_hbm, ew_hbm, nk, out_hbm):
        pltpu.sync_copy(z128.at[pl.ds(s * SLAB, SLAB)],
                        acc.at[pl.ds(s * SLAB, SLAB)])
        plsc.subcore_barrier()

        def body(j, _):
            k = w + j * NW

            @pl.when(k < nk)
            def _():
                pltpu.sync_copy(src_hbm.at[pl.ds(k * C, C)], srcbuf)
                pltpu.sync_copy(dst_hbm.at[pl.ds(k * C, C)], dstbuf)
                pltpu.sync_copy(ew_hbm.at[pl.ds(k * C, C)], ewbuf)
                pltpu.async_copy(h_hbm.at[srcbuf], rows, sem).wait()
                for t in range(C // 16):
                    sv = srcbuf[pl.ds(t * 16, 16)]
                    dinvv = plsc.load_gather(dinvbuf, [sv])
                    wv = ewbuf[pl.ds(t * 16, 16)]
                    normbuf[pl.ds(t * 16, 16)] = dinvv * wv

                def scale(e, _):
                    nspl = plsc.load_gather(normbuf, [lax.broadcast(e, (16,))])
                    for f in range(F // 16):
                        rows[e, pl.ds(f * 16, 16)] = (
                            rows[e, pl.ds(f * 16, 16)] * nspl)
                    return 0

                lax.fori_loop(0, C, scale, 0)
                pltpu.sync_copy(rows, acc.at[dstbuf], add=True)

            return 0

        lax.fori_loop(0, (nk + NW - 1) // NW, body, 0)
        plsc.subcore_barrier()
        pltpu.sync_copy(acc.at[pl.ds(s * SLAB, SLAB)],
                        out_hbm.at[c, pl.ds(s * SLAB, SLAB)])
        plsc.subcore_barrier()

    compute_dinv(degp_l)
    graph_pass(h_l, src_l, dst_l, ew_l, 320000 // C, accp_l)
    compute_dinv(degp_s)
    graph_pass(h_s, src_s, dst_s, ew_s, 160000 // C, accp_s)


def _head_body(accp_l_ref, accp_s_ref, degp_l_ref, degp_s_ref,
               batch_l_ref, batch_s_ref, b1a_ref, b1b_ref,
               wl1_ref, bl1_ref, wl2_ref, bl2_ref,
               gamma_ref, beta_ref, wo_ref, bo_ref,
               out_ref, hidden_ref):
    def pooled(accp_ref, degp_ref, batch_ref, b_ref):
        acc = accp_ref[0] + accp_ref[1]
        deg = degp_ref[0] + degp_ref[1]
        dinv = jnp.where(deg > 0, lax.rsqrt(deg), 0.0)
        node = jnp.maximum(acc * dinv + b_ref[...], 0.0)
        iota = lax.broadcasted_iota(jnp.int32, (G, NP), 0)
        pt = (iota == batch_ref[...]).astype(jnp.float32)
        seg = jnp.dot(pt, node, preferred_element_type=jnp.float32,
                      precision=lax.Precision.HIGHEST)
        cnt = jnp.sum(pt, axis=1, keepdims=True)
        return seg / jnp.maximum(cnt, 1.0)

    h1 = pooled(accp_l_ref, degp_l_ref, batch_l_ref, b1a_ref)
    h2 = pooled(accp_s_ref, degp_s_ref, batch_s_ref, b1b_ref)
    hid = jnp.concatenate([h1, h2], axis=1)
    hid = jnp.dot(hid, wl1_ref[...],
                  preferred_element_type=jnp.float32) + bl1_ref[...]
    hid = jnp.dot(hid, wl2_ref[...],
                  preferred_element_type=jnp.float32) + bl2_ref[...]
    mean = jnp.mean(hid, axis=0, keepdims=True)
    var = jnp.mean((hid - mean) ** 2, axis=0, keepdims=True)
    hid = gamma_ref[...] * (hid - mean) / jnp.sqrt(var + 1e-5) + beta_ref[...]
    hid = jnp.maximum(hid, 0.0)
    hidden_ref[...] = hid
    out_ref[...] = jnp.dot(hid, wo_ref[...],
                           preferred_element_type=jnp.float32) + bo_ref[...]


def _pad_deg(degp_flat):
    return degp_flat.reshape(NC, NP, 1)


def _pad_batch(batch):
    return jnp.pad(batch, (0, NP - N), constant_values=-1).reshape(1, NP)


def kernel(x_l, edge_index_l, edge_weight_l, x_s, edge_index_s, edge_weight_s,
           batch_index_l, batch_index_s,
           W1a, b1a, W1b, b1b, Wl1, bl1, Wl2, bl2, gamma, beta, Wo, bo):
    src_l, dst_l = edge_index_l[0], edge_index_l[1]
    src_s, dst_s = edge_index_s[0], edge_index_s[1]
    z1 = jnp.zeros((NP,), jnp.float32)
    z128 = jnp.zeros((NP, F), jnp.float32)

    degp_l, degp_s = _deg_kernel(dst_l, edge_weight_l, dst_s, edge_weight_s,
                                 z1)

    h_l, h_s = pl.pallas_call(
        _mm_body,
        out_shape=(jax.ShapeDtypeStruct((N, F), jnp.float32),
                   jax.ShapeDtypeStruct((N, F), jnp.float32)),
    )(x_l, W1a, x_s, W1b)

    accp_l, accp_s = _msg_kernel(h_l, h_s, degp_l, degp_s,
                                 src_l, dst_l, edge_weight_l,
                                 src_s, dst_s, edge_weight_s, z128)

    out, hidden = pl.pallas_call(
        _head_body,
        out_shape=(jax.ShapeDtypeStruct((G, 1), jnp.float32),
                   jax.ShapeDtypeStruct((G, F), jnp.float32)),
    )(accp_l, accp_s,
      _pad_deg(degp_l), _pad_deg(degp_s),
      _pad_batch(batch_index_l), _pad_batch(batch_index_s),
      b1a.reshape(1, F), b1b.reshape(1, F),
      Wl1, bl1.reshape(1, 2 * F), Wl2, bl2.reshape(1, F),
      gamma.reshape(1, F), beta.reshape(1, F), Wo, bo.reshape(1, 1))
    return (out, hidden)
